# baseline (speedup 1.0000x reference)
import functools

import jax
import jax.numpy as jnp
import numpy as np
from jax import lax
from jax.experimental import pallas as pl
from jax.experimental.pallas import tpu as pltpu
from jax.experimental.pallas import tpu_sc as plsc

COORDS_RANGE = 15.0
NORM_FACTOR = 100.0

NC = 2
NS = 16
NW = NC * NS


def _precompute_body(h_ref, w1a_ref, w1b_ref, b1_ref, a_ref, b_ref):
    hx = h_ref[...]
    a_ref[...] = jnp.dot(hx, w1a_ref[...], preferred_element_type=jnp.float32) + b1_ref[...]
    b_ref[...] = jnp.dot(hx, w1b_ref[...], preferred_element_type=jnp.float32)


def _precompute_tables(h, w1a, w1b, b1r):
    n, hdim = h.shape
    bn = 2000
    grid = pl.cdiv(n, bn)
    return pl.pallas_call(
        _precompute_body,
        grid=(grid,),
        in_specs=[
            pl.BlockSpec((bn, hdim), lambda i: (i, 0)),
            pl.BlockSpec((hdim, hdim), lambda i: (0, 0)),
            pl.BlockSpec((hdim, hdim), lambda i: (0, 0)),
            pl.BlockSpec((1, hdim), lambda i: (0, 0)),
        ],
        out_specs=[
            pl.BlockSpec((bn, hdim), lambda i: (i, 0)),
            pl.BlockSpec((bn, hdim), lambda i: (i, 0)),
        ],
        out_shape=[
            jax.ShapeDtypeStruct((n, hdim), jnp.float32),
            jax.ShapeDtypeStruct((n, hdim), jnp.float32),
        ],
    )(h, w1a, w1b, b1r)


def _sc_gather(row, col, atab, btab):
    e = row.shape[0]
    hdim = atab.shape[1]
    per_w = e // NW
    ch = 80
    assert per_w % ch == 0
    nch = per_w // ch
    nbuf = 3
    mesh = plsc.VectorSubcoreMesh(core_axis_name="c", subcore_axis_name="s")

    @functools.partial(
        pl.kernel,
        out_type=jax.ShapeDtypeStruct((e, hdim), jnp.float32),
        mesh=mesh,
        compiler_params=pltpu.CompilerParams(needs_layout_passes=False),
        scratch_types=[
            pltpu.VMEM((per_w,), jnp.int32),
            pltpu.VMEM((per_w,), jnp.int32),
        ] + [pltpu.VMEM((ch, hdim), jnp.float32)] * (3 * nbuf)
          + [pltpu.SemaphoreType.DMA] * (2 * nbuf),
    )
    def gather_k(row_hbm, col_hbm, a_hbm, b_hbm, out_hbm, ri_v, ci_v, *rest):
        datbufs = rest[:3 * nbuf]
        sems = rest[3 * nbuf:]
        bufs = tuple(
            (datbufs[3 * b], datbufs[3 * b + 1], datbufs[3 * b + 2],
             sems[2 * b], sems[2 * b + 1])
            for b in range(nbuf))
        wid = lax.axis_index("s") * NC + lax.axis_index("c")
        base = wid * per_w

        pltpu.sync_copy(row_hbm.at[pl.ds(base, per_w)], ri_v)
        pltpu.sync_copy(col_hbm.at[pl.ds(base, per_w)], ci_v)

        def fire_gathers(i, pb):
            ba, bb, _, semg, _ = pb
            pltpu.async_copy(a_hbm.at[ri_v.at[pl.ds(i * ch, ch)]], ba, semg)
            pltpu.async_copy(b_hbm.at[ci_v.at[pl.ds(i * ch, ch)]], bb, semg)

        def process(i, b):
            ba, bb, bo, semg, semo = bufs[b]
            @pl.when(i >= nbuf)
            def _():
                pltpu.make_async_copy(bo, out_hbm.at[pl.ds(0, ch)], semo).wait()

            pltpu.make_async_copy(a_hbm.at[ri_v.at[pl.ds(i * ch, ch)]], ba, semg).wait()
            pltpu.make_async_copy(b_hbm.at[ci_v.at[pl.ds(i * ch, ch)]], bb, semg).wait()

            @pl.when(i + nbuf - 1 < nch)
            def _():
                fire_gathers(i + nbuf - 1, bufs[(b + nbuf - 1) % nbuf])

            def add_body(ei, c2):
                for k in range(hdim // 16):
                    sl = pl.ds(k * 16, 16)
                    bo[ei, sl] = ba[ei, sl] + bb[ei, sl]
                return c2

            lax.fori_loop(0, ch, add_body, 0)
            off = pl.multiple_of(base + i * ch, 8)
            pltpu.async_copy(bo, out_hbm.at[pl.ds(off, ch)], semo)

        for j in range(nbuf - 1):
            fire_gathers(j, bufs[j])

        def outer(io, c):
            for b in range(nbuf):
                process(nbuf * io + b, b)
            return c

        lax.fori_loop(0, nch // nbuf, outer, 0)
        for j in range(nch - nch % nbuf, nch):
            process(j, j % nbuf)
        for b in range(min(nbuf, nch)):
            pltpu.make_async_copy(bufs[b][2], out_hbm.at[pl.ds(0, ch)],
                                  bufs[b][4]).wait()

    return gather_k(row, col, atab, btab)


def _mlp_body(g_ref, ea_ref, w1e_ref, w2_ref, b2_ref, w3r_ref, t_ref):
    g = g_ref[...]
    x1 = g + ea_ref[...] * w1e_ref[...]
    x1 = x1 * jax.nn.sigmoid(x1)
    x2 = jnp.dot(x1.astype(jnp.bfloat16), w2_ref[...].astype(jnp.bfloat16),
                 preferred_element_type=jnp.float32) + b2_ref[...]
    x2 = x2 * jax.nn.sigmoid(x2)
    phi = lax.dot_general(w3r_ref[...], x2, (((1,), (1,)), ((), ())),
                          preferred_element_type=jnp.float32)
    t_ref[0] = jnp.tanh(phi) * (COORDS_RANGE / NORM_FACTOR)


def _mlp_edges(g, ea_full, lo, w1e, w2, b2r, w3r):
    e, hdim = g.shape
    be = 2560
    grid = e // be
    nb = grid
    lob = lo // be
    return pl.pallas_call(
        _mlp_body,
        grid=(grid,),
        in_specs=[
            pl.BlockSpec((be, hdim), lambda i: (i, 0)),
            pl.BlockSpec((be, 1), lambda i: (i + lob, 0)),
            pl.BlockSpec((1, hdim), lambda i: (0, 0)),
            pl.BlockSpec((hdim, hdim), lambda i: (0, 0)),
            pl.BlockSpec((1, hdim), lambda i: (0, 0)),
            pl.BlockSpec((1, hdim), lambda i: (0, 0)),
        ],
        out_specs=[
            pl.BlockSpec((1, 1, be), lambda i: (i, 0, 0)),
        ],
        out_shape=[
            jax.ShapeDtypeStruct((nb, 1, be), jnp.float32),
        ],
    )(g, ea_full, w1e, w2, b2r, w3r)


def _sc_scatter(row, tphi, cd, coord_planar, npad, half_f):
    e = row.shape[0]
    per_tile = e // NS
    ce = 400
    nch = per_tile // ce
    ngrp = 3 * ce // 16
    rows = half_f // 128
    rows_t = rows // NS
    slice_f = rows_t * 128
    mesh = plsc.VectorSubcoreMesh(core_axis_name="c", subcore_axis_name="s")

    @functools.partial(
        pl.kernel,
        out_type=jax.ShapeDtypeStruct((NC * half_f,), jnp.float32),
        mesh=mesh,
        compiler_params=pltpu.CompilerParams(needs_layout_passes=False),
        scratch_types=[
            pltpu.VMEM((rows, 128), jnp.float32),
            pltpu.VMEM((ce,), jnp.int32),
            pltpu.VMEM((ce,), jnp.float32),
            pltpu.VMEM((ce, 3), jnp.float32),
            pltpu.VMEM((NS, rows_t, 128), jnp.float32),
            pltpu.VMEM((slice_f,), jnp.float32),
            pltpu.VMEM((slice_f,), jnp.float32),
            pltpu.VMEM_SHARED((NS, rows, 128), jnp.float32),
        ],
    )
    def scatter_k(row_hbm, t_hbm, cd_hbm, coord_hbm, out_hbm,
                  acc, ibuf, tbuf, cdbuf, redbuf, cbuf, obuf, shared):
        cid = lax.axis_index("c")
        sid = lax.axis_index("s")
        lo = cid * half_f
        iota = lax.iota(jnp.int32, 16)

        def zero_body(i, c):
            for k in range(8):
                acc[i, pl.ds(k * 16, 16)] = jnp.zeros((16,), jnp.float32)
            return c

        lax.fori_loop(0, rows, zero_body, 0)

        def chunk_body(i, c):
            off = sid * per_tile + i * ce
            pltpu.sync_copy(row_hbm.at[pl.ds(off, ce)], ibuf)
            pltpu.sync_copy(t_hbm.at[pl.ds(off, ce)], tbuf)
            pltpu.sync_copy(cd_hbm.at[pl.ds(off, ce), :], cdbuf)

            def grp_body(gi, c2):
                p = gi * 16 + iota
                el = lax.shift_right_logical(p * 21846, 16)
                comp = p - el * 3
                cdv = plsc.load_gather(cdbuf, [el, comp])
                tv = plsc.load_gather(tbuf, [el])
                rv = plsc.load_gather(ibuf, [el])
                val = cdv * tv
                pv = rv + comp * npad - lo
                inb = (pv >= 0) & (pv < half_f)
                pvc = jnp.where(inb, pv, 0)
                hi = lax.shift_right_logical(pvc, 7)
                lo7 = jnp.bitwise_and(pvc, 127)
                plsc.addupdate_scatter(acc, [hi, lo7], val, mask=inb)
                return c2

            lax.fori_loop(0, ngrp, grp_body, 0)
            return c

        lax.fori_loop(0, nch, chunk_body, 0)

        plsc.subcore_barrier()
        pltpu.sync_copy(acc, shared.at[sid])
        plsc.subcore_barrier()

        pltpu.sync_copy(shared.at[:, pl.ds(sid * rows_t, rows_t), :], redbuf)
        pltpu.sync_copy(coord_hbm.at[pl.ds(lo + sid * slice_f, slice_f)], cbuf)

        def red_body(gi, c):
            r = lax.shift_right_logical(gi, 3)
            k = jnp.bitwise_and(gi, 7)
            sl = pl.ds(k * 16, 16)
            fl = pl.ds(r * 128 + k * 16, 16)
            s = cbuf[fl]
            for t in range(NS):
                s = s + redbuf[t, r, sl]
            obuf[fl] = s
            return c

        lax.fori_loop(0, rows_t * 8, red_body, 0)
        pltpu.sync_copy(obuf, out_hbm.at[pl.ds(lo + sid * slice_f, slice_f)])

    return scatter_k(row, tphi, cd, coord_planar)


def kernel(h, coord, edge_index, coord_diff, coord_cross, edge_attr,
           w1, b1, w2, b2, w3):
    n, hdim = h.shape
    e = edge_index.shape[1]

    row = edge_index[0]
    col = edge_index[1]

    w1a = w1[:hdim]
    w1b = w1[hdim:2 * hdim]
    w1e = w1[2 * hdim:2 * hdim + 1]
    b1r = b1.reshape(1, hdim)
    b2r = b2.reshape(1, hdim)
    w3r = w3.reshape(1, hdim)

    atab, btab = _precompute_tables(h, w1a, w1b, b1r)

    nblk = e // 2560
    cuts = [0, (nblk // 3) * 2560, (2 * nblk // 3) * 2560, e]
    gs, ts = [], []
    for s in range(len(cuts) - 1):
        lo, hi = cuts[s], cuts[s + 1]
        gs.append(_sc_gather(row[lo:hi], col[lo:hi], atab, btab))
    for s in range(len(cuts) - 1):
        lo, hi = cuts[s], cuts[s + 1]
        ts.append(_mlp_edges(gs[s], edge_attr, lo, w1e, w2, b2r, w3r)[0])
    tphi = jnp.concatenate(ts).reshape(-1)

    npad = 10240
    half_f = 16384
    coord_planar = jnp.pad(coord.T, ((0, 0), (0, npad - n))).reshape(-1)
    coord_planar = jnp.pad(coord_planar, (0, NC * half_f - 3 * npad))
    out_flat = _sc_scatter(row, tphi, coord_diff, coord_planar, npad, half_f)
    return out_flat[:3 * npad].reshape(3, npad)[:, :n].T


# speedup vs baseline: 1.4852x; 1.4852x over previous
import functools

import jax
import jax.numpy as jnp
import numpy as np
from jax import lax
from jax.experimental import pallas as pl
from jax.experimental.pallas import tpu as pltpu
from jax.experimental.pallas import tpu_sc as plsc

COORDS_RANGE = 15.0
NORM_FACTOR = 100.0

NC = 2
NS = 16
NW = NC * NS


def _precompute_body(h_ref, w1a_ref, w1b_ref, b1_ref, a_ref, b_ref):
    hx = h_ref[...]
    a_ref[...] = jnp.dot(hx, w1a_ref[...], preferred_element_type=jnp.float32) + b1_ref[...]
    b_ref[...] = jnp.dot(hx, w1b_ref[...], preferred_element_type=jnp.float32)


def _precompute_tables(h, w1a, w1b, b1r):
    n, hdim = h.shape
    bn = 2000
    grid = pl.cdiv(n, bn)
    return pl.pallas_call(
        _precompute_body,
        grid=(grid,),
        in_specs=[
            pl.BlockSpec((bn, hdim), lambda i: (i, 0)),
            pl.BlockSpec((hdim, hdim), lambda i: (0, 0)),
            pl.BlockSpec((hdim, hdim), lambda i: (0, 0)),
            pl.BlockSpec((1, hdim), lambda i: (0, 0)),
        ],
        out_specs=[
            pl.BlockSpec((bn, hdim), lambda i: (i, 0)),
            pl.BlockSpec((bn, hdim), lambda i: (i, 0)),
        ],
        out_shape=[
            jax.ShapeDtypeStruct((n, hdim), jnp.float32),
            jax.ShapeDtypeStruct((n, hdim), jnp.float32),
        ],
    )(h, w1a, w1b, b1r)


def _sc_gather(row, col, atab, btab):
    e = row.shape[0]
    hdim = atab.shape[1]
    per_w = e // NW
    ch = 80
    assert per_w % ch == 0
    nch = per_w // ch
    nbuf = 3
    mesh = plsc.VectorSubcoreMesh(core_axis_name="c", subcore_axis_name="s")

    @functools.partial(
        pl.kernel,
        out_type=jax.ShapeDtypeStruct((e, hdim), jnp.float32),
        mesh=mesh,
        compiler_params=pltpu.CompilerParams(needs_layout_passes=False),
        scratch_types=[
            pltpu.VMEM((per_w,), jnp.int32),
            pltpu.VMEM((per_w,), jnp.int32),
        ] + [pltpu.VMEM((ch, hdim), jnp.float32)] * (3 * nbuf)
          + [pltpu.SemaphoreType.DMA] * (2 * nbuf),
    )
    def gather_k(row_hbm, col_hbm, a_hbm, b_hbm, out_hbm, ri_v, ci_v, *rest):
        datbufs = rest[:3 * nbuf]
        sems = rest[3 * nbuf:]
        bufs = tuple(
            (datbufs[3 * b], datbufs[3 * b + 1], datbufs[3 * b + 2],
             sems[2 * b], sems[2 * b + 1])
            for b in range(nbuf))
        wid = lax.axis_index("s") * NC + lax.axis_index("c")
        base = wid * per_w

        pltpu.sync_copy(row_hbm.at[pl.ds(base, per_w)], ri_v)
        pltpu.sync_copy(col_hbm.at[pl.ds(base, per_w)], ci_v)

        def fire_gathers(i, pb):
            ba, bb, _, semg, _ = pb
            pltpu.async_copy(a_hbm.at[ri_v.at[pl.ds(i * ch, ch)]], ba, semg)
            pltpu.async_copy(b_hbm.at[ci_v.at[pl.ds(i * ch, ch)]], bb, semg)

        def process(i, b):
            ba, bb, bo, semg, semo = bufs[b]
            @pl.when(i >= nbuf)
            def _():
                pltpu.make_async_copy(bo, out_hbm.at[pl.ds(0, ch)], semo).wait()

            pltpu.make_async_copy(a_hbm.at[ri_v.at[pl.ds(i * ch, ch)]], ba, semg).wait()
            pltpu.make_async_copy(b_hbm.at[ci_v.at[pl.ds(i * ch, ch)]], bb, semg).wait()

            @pl.when(i + nbuf - 1 < nch)
            def _():
                fire_gathers(i + nbuf - 1, bufs[(b + nbuf - 1) % nbuf])

            def add_body(ei, c2):
                for k in range(hdim // 16):
                    sl = pl.ds(k * 16, 16)
                    bo[ei, sl] = ba[ei, sl] + bb[ei, sl]
                return c2

            lax.fori_loop(0, ch, add_body, 0)
            off = pl.multiple_of(base + i * ch, 8)
            pltpu.async_copy(bo, out_hbm.at[pl.ds(off, ch)], semo)

        for j in range(nbuf - 1):
            fire_gathers(j, bufs[j])

        def outer(io, c):
            for b in range(nbuf):
                process(nbuf * io + b, b)
            return c

        lax.fori_loop(0, nch // nbuf, outer, 0)
        for j in range(nch - nch % nbuf, nch):
            process(j, j % nbuf)
        for b in range(min(nbuf, nch)):
            pltpu.make_async_copy(bufs[b][2], out_hbm.at[pl.ds(0, ch)],
                                  bufs[b][4]).wait()

    return gather_k(row, col, atab, btab)


def _mlp_body(g_ref, ea_ref, cdt_ref, w1e_ref, w2_ref, b2_ref, w3r_ref,
              tx_ref, ty_ref, tz_ref):
    g = g_ref[...]
    x1 = g + ea_ref[...] * w1e_ref[...]
    x1 = x1 * jax.nn.sigmoid(x1)
    x2 = jnp.dot(x1.astype(jnp.bfloat16), w2_ref[...].astype(jnp.bfloat16),
                 preferred_element_type=jnp.float32) + b2_ref[...]
    x2 = x2 * jax.nn.sigmoid(x2)
    phi = lax.dot_general(w3r_ref[...], x2, (((1,), (1,)), ((), ())),
                          preferred_element_type=jnp.float32)
    t = jnp.tanh(phi) * (COORDS_RANGE / NORM_FACTOR)
    cdt = cdt_ref[...]
    tx_ref[0] = cdt[0:1, :] * t
    ty_ref[0] = cdt[1:2, :] * t
    tz_ref[0] = cdt[2:3, :] * t


def _mlp_edges(g, ea, cdt, w1e, w2, b2r, w3r):
    e, hdim = g.shape
    be = 2560
    grid = e // be
    nb = grid
    return pl.pallas_call(
        _mlp_body,
        grid=(grid,),
        in_specs=[
            pl.BlockSpec((be, hdim), lambda i: (i, 0)),
            pl.BlockSpec((be, 1), lambda i: (i, 0)),
            pl.BlockSpec((3, be), lambda i: (0, i)),
            pl.BlockSpec((1, hdim), lambda i: (0, 0)),
            pl.BlockSpec((hdim, hdim), lambda i: (0, 0)),
            pl.BlockSpec((1, hdim), lambda i: (0, 0)),
            pl.BlockSpec((1, hdim), lambda i: (0, 0)),
        ],
        out_specs=[
            pl.BlockSpec((1, 1, be), lambda i: (i, 0, 0)),
            pl.BlockSpec((1, 1, be), lambda i: (i, 0, 0)),
            pl.BlockSpec((1, 1, be), lambda i: (i, 0, 0)),
        ],
        out_shape=[
            jax.ShapeDtypeStruct((nb, 1, be), jnp.float32),
            jax.ShapeDtypeStruct((nb, 1, be), jnp.float32),
            jax.ShapeDtypeStruct((nb, 1, be), jnp.float32),
        ],
    )(g, ea, cdt, w1e, w2, b2r, w3r)


def _sc_scatter(row, tx, ty, tz, coord_planar, npad, half_f):
    e = row.shape[0]
    per_tile = e // NS
    ce = 2000
    nch = per_tile // ce
    ngrp = ce // 16
    rows = half_f // 128
    rows_t = rows // NS
    slice_f = rows_t * 128
    mesh = plsc.VectorSubcoreMesh(core_axis_name="c", subcore_axis_name="s")

    @functools.partial(
        pl.kernel,
        out_type=jax.ShapeDtypeStruct((NC * half_f,), jnp.float32),
        mesh=mesh,
        compiler_params=pltpu.CompilerParams(needs_layout_passes=False),
        scratch_types=[
            pltpu.VMEM((rows, 128), jnp.float32),
            pltpu.VMEM((ce,), jnp.int32),
            pltpu.VMEM((ce,), jnp.float32),
            pltpu.VMEM((ce,), jnp.float32),
            pltpu.VMEM((ce,), jnp.float32),
            pltpu.VMEM((NS, rows_t, 128), jnp.float32),
            pltpu.VMEM((slice_f,), jnp.float32),
            pltpu.VMEM((slice_f,), jnp.float32),
            pltpu.VMEM_SHARED((NS, rows, 128), jnp.float32),
        ],
    )
    def scatter_k(row_hbm, tx_hbm, ty_hbm, tz_hbm, coord_hbm, out_hbm,
                  acc, ibuf, xbuf, ybuf, zbuf, redbuf, cbuf, obuf, shared):
        cid = lax.axis_index("c")
        sid = lax.axis_index("s")
        lo = cid * half_f

        def zero_body(i, c):
            for k in range(8):
                acc[i, pl.ds(k * 16, 16)] = jnp.zeros((16,), jnp.float32)
            return c

        lax.fori_loop(0, rows, zero_body, 0)

        def chunk_body(i, c):
            off = sid * per_tile + i * ce
            pltpu.sync_copy(row_hbm.at[pl.ds(off, ce)], ibuf)
            pltpu.sync_copy(tx_hbm.at[pl.ds(off, ce)], xbuf)
            pltpu.sync_copy(ty_hbm.at[pl.ds(off, ce)], ybuf)
            pltpu.sync_copy(tz_hbm.at[pl.ds(off, ce)], zbuf)

            def grp_body(gi, c2):
                sl = pl.ds(gi * 16, 16)
                iv = ibuf[sl]
                for ci, buf in ((0, xbuf), (1, ybuf), (2, zbuf)):
                    pv = iv + (ci * npad - lo)
                    inb = (pv >= 0) & (pv < half_f)
                    pvc = jnp.where(inb, pv, 0)
                    hi = lax.shift_right_logical(pvc, 7)
                    lo7 = jnp.bitwise_and(pvc, 127)
                    plsc.addupdate_scatter(acc, [hi, lo7], buf[sl], mask=inb)
                return c2

            lax.fori_loop(0, ngrp, grp_body, 0)
            return c

        lax.fori_loop(0, nch, chunk_body, 0)

        plsc.subcore_barrier()
        pltpu.sync_copy(acc, shared.at[sid])
        plsc.subcore_barrier()

        pltpu.sync_copy(shared.at[:, pl.ds(sid * rows_t, rows_t), :], redbuf)
        pltpu.sync_copy(coord_hbm.at[pl.ds(lo + sid * slice_f, slice_f)], cbuf)

        def red_body(gi, c):
            r = lax.shift_right_logical(gi, 3)
            k = jnp.bitwise_and(gi, 7)
            sl = pl.ds(k * 16, 16)
            fl = pl.ds(r * 128 + k * 16, 16)
            s = cbuf[fl]
            for t in range(NS):
                s = s + redbuf[t, r, sl]
            obuf[fl] = s
            return c

        lax.fori_loop(0, rows_t * 8, red_body, 0)
        pltpu.sync_copy(obuf, out_hbm.at[pl.ds(lo + sid * slice_f, slice_f)])

    return scatter_k(row, tx, ty, tz, coord_planar)


def kernel(h, coord, edge_index, coord_diff, coord_cross, edge_attr,
           w1, b1, w2, b2, w3):
    n, hdim = h.shape
    e = edge_index.shape[1]

    row = edge_index[0]
    col = edge_index[1]

    w1a = w1[:hdim]
    w1b = w1[hdim:2 * hdim]
    w1e = w1[2 * hdim:2 * hdim + 1]
    b1r = b1.reshape(1, hdim)
    b2r = b2.reshape(1, hdim)
    w3r = w3.reshape(1, hdim)

    atab, btab = _precompute_tables(h, w1a, w1b, b1r)

    cdt = coord_diff.T

    nblk = e // 2560
    cuts = [0, (nblk // 3) * 2560, (2 * nblk // 3) * 2560, e]
    gs, ts = [], []
    for s in range(len(cuts) - 1):
        lo, hi = cuts[s], cuts[s + 1]
        gs.append(_sc_gather(row[lo:hi], col[lo:hi], atab, btab))
    for s in range(len(cuts) - 1):
        lo, hi = cuts[s], cuts[s + 1]
        ts.append(_mlp_edges(gs[s], edge_attr[lo:hi], cdt[:, lo:hi],
                             w1e, w2, b2r, w3r))
    trans_x = jnp.concatenate([t[0] for t in ts])
    trans_y = jnp.concatenate([t[1] for t in ts])
    trans_z = jnp.concatenate([t[2] for t in ts])

    npad = 10240
    half_f = 16384
    coord_planar = jnp.pad(coord.T, ((0, 0), (0, npad - n))).reshape(-1)
    coord_planar = jnp.pad(coord_planar, (0, NC * half_f - 3 * npad))
    out_flat = _sc_scatter(row, trans_x.reshape(-1), trans_y.reshape(-1),
                           trans_z.reshape(-1), coord_planar, npad, half_f)
    return out_flat[:3 * npad].reshape(3, npad)[:, :n].T


# speedup vs baseline: 1.4866x; 1.0010x over previous
import functools

import jax
import jax.numpy as jnp
import numpy as np
from jax import lax
from jax.experimental import pallas as pl
from jax.experimental.pallas import tpu as pltpu
from jax.experimental.pallas import tpu_sc as plsc

COORDS_RANGE = 15.0
NORM_FACTOR = 100.0

NC = 2
NS = 16
NW = NC * NS


def _precompute_body(h_ref, w1a_ref, w1b_ref, b1_ref, a_ref, b_ref):
    hx = h_ref[...]
    a_ref[...] = jnp.dot(hx, w1a_ref[...], preferred_element_type=jnp.float32) + b1_ref[...]
    b_ref[...] = jnp.dot(hx, w1b_ref[...], preferred_element_type=jnp.float32)


def _precompute_tables(h, w1a, w1b, b1r):
    n, hdim = h.shape
    bn = 2000
    grid = pl.cdiv(n, bn)
    return pl.pallas_call(
        _precompute_body,
        grid=(grid,),
        in_specs=[
            pl.BlockSpec((bn, hdim), lambda i: (i, 0)),
            pl.BlockSpec((hdim, hdim), lambda i: (0, 0)),
            pl.BlockSpec((hdim, hdim), lambda i: (0, 0)),
            pl.BlockSpec((1, hdim), lambda i: (0, 0)),
        ],
        out_specs=[
            pl.BlockSpec((bn, hdim), lambda i: (i, 0)),
            pl.BlockSpec((bn, hdim), lambda i: (i, 0)),
        ],
        out_shape=[
            jax.ShapeDtypeStruct((n, hdim), jnp.float32),
            jax.ShapeDtypeStruct((n, hdim), jnp.float32),
        ],
    )(h, w1a, w1b, b1r)


def _sc_gather(row, col, atab, btab):
    e = row.shape[0]
    hdim = atab.shape[1]
    per_w = e // NW
    ch = 80
    assert per_w % ch == 0
    nch = per_w // ch
    nbuf = 3
    mesh = plsc.VectorSubcoreMesh(core_axis_name="c", subcore_axis_name="s")

    @functools.partial(
        pl.kernel,
        out_type=jax.ShapeDtypeStruct((e, hdim), jnp.float32),
        mesh=mesh,
        compiler_params=pltpu.CompilerParams(needs_layout_passes=False),
        scratch_types=[
            pltpu.VMEM((per_w,), jnp.int32),
            pltpu.VMEM((per_w,), jnp.int32),
        ] + [pltpu.VMEM((ch, hdim), jnp.float32)] * (3 * nbuf)
          + [pltpu.SemaphoreType.DMA] * (2 * nbuf),
    )
    def gather_k(row_hbm, col_hbm, a_hbm, b_hbm, out_hbm, ri_v, ci_v, *rest):
        datbufs = rest[:3 * nbuf]
        sems = rest[3 * nbuf:]
        bufs = tuple(
            (datbufs[3 * b], datbufs[3 * b + 1], datbufs[3 * b + 2],
             sems[2 * b], sems[2 * b + 1])
            for b in range(nbuf))
        wid = lax.axis_index("s") * NC + lax.axis_index("c")
        base = wid * per_w

        pltpu.sync_copy(row_hbm.at[pl.ds(base, per_w)], ri_v)
        pltpu.sync_copy(col_hbm.at[pl.ds(base, per_w)], ci_v)

        def fire_gathers(i, pb):
            ba, bb, _, semg, _ = pb
            pltpu.async_copy(a_hbm.at[ri_v.at[pl.ds(i * ch, ch)]], ba, semg)
            pltpu.async_copy(b_hbm.at[ci_v.at[pl.ds(i * ch, ch)]], bb, semg)

        def process(i, b):
            ba, bb, bo, semg, semo = bufs[b]
            @pl.when(i >= nbuf)
            def _():
                pltpu.make_async_copy(bo, out_hbm.at[pl.ds(0, ch)], semo).wait()

            pltpu.make_async_copy(a_hbm.at[ri_v.at[pl.ds(i * ch, ch)]], ba, semg).wait()
            pltpu.make_async_copy(b_hbm.at[ci_v.at[pl.ds(i * ch, ch)]], bb, semg).wait()

            @pl.when(i + nbuf - 1 < nch)
            def _():
                fire_gathers(i + nbuf - 1, bufs[(b + nbuf - 1) % nbuf])

            def add_body(ei, c2):
                for k in range(hdim // 16):
                    sl = pl.ds(k * 16, 16)
                    bo[ei, sl] = ba[ei, sl] + bb[ei, sl]
                return c2

            lax.fori_loop(0, ch, add_body, 0)
            off = pl.multiple_of(base + i * ch, 8)
            pltpu.async_copy(bo, out_hbm.at[pl.ds(off, ch)], semo)

        for j in range(nbuf - 1):
            fire_gathers(j, bufs[j])

        def outer(io, c):
            for b in range(nbuf):
                process(nbuf * io + b, b)
            return c

        lax.fori_loop(0, nch // nbuf, outer, 0)
        for j in range(nch - nch % nbuf, nch):
            process(j, j % nbuf)
        for b in range(min(nbuf, nch)):
            pltpu.make_async_copy(bufs[b][2], out_hbm.at[pl.ds(0, ch)],
                                  bufs[b][4]).wait()

    return gather_k(row, col, atab, btab)


def _mlp_body(g_ref, ea_ref, cdt_ref, w1e_ref, w2_ref, b2_ref, w3r_ref,
              tx_ref, ty_ref, tz_ref):
    g = g_ref[...]
    x1 = g + ea_ref[...] * w1e_ref[...]
    x1 = x1 * jax.nn.sigmoid(x1)
    x2 = jnp.dot(x1.astype(jnp.bfloat16), w2_ref[...].astype(jnp.bfloat16),
                 preferred_element_type=jnp.float32) + b2_ref[...]
    x2 = x2 * jax.nn.sigmoid(x2)
    phi = lax.dot_general(w3r_ref[...], x2, (((1,), (1,)), ((), ())),
                          preferred_element_type=jnp.float32)
    t = jnp.tanh(phi) * (COORDS_RANGE / NORM_FACTOR)
    cdt = cdt_ref[...]
    tx_ref[0] = cdt[0:1, :] * t
    ty_ref[0] = cdt[1:2, :] * t
    tz_ref[0] = cdt[2:3, :] * t


def _mlp_edges(g, ea, cdt, w1e, w2, b2r, w3r):
    e, hdim = g.shape
    be = 2560
    grid = e // be
    nb = grid
    return pl.pallas_call(
        _mlp_body,
        grid=(grid,),
        in_specs=[
            pl.BlockSpec((be, hdim), lambda i: (i, 0)),
            pl.BlockSpec((be, 1), lambda i: (i, 0)),
            pl.BlockSpec((3, be), lambda i: (0, i)),
            pl.BlockSpec((1, hdim), lambda i: (0, 0)),
            pl.BlockSpec((hdim, hdim), lambda i: (0, 0)),
            pl.BlockSpec((1, hdim), lambda i: (0, 0)),
            pl.BlockSpec((1, hdim), lambda i: (0, 0)),
        ],
        out_specs=[
            pl.BlockSpec((1, 1, be), lambda i: (i, 0, 0)),
            pl.BlockSpec((1, 1, be), lambda i: (i, 0, 0)),
            pl.BlockSpec((1, 1, be), lambda i: (i, 0, 0)),
        ],
        out_shape=[
            jax.ShapeDtypeStruct((nb, 1, be), jnp.float32),
            jax.ShapeDtypeStruct((nb, 1, be), jnp.float32),
            jax.ShapeDtypeStruct((nb, 1, be), jnp.float32),
        ],
    )(g, ea, cdt, w1e, w2, b2r, w3r)


def _sc_scatter(row, tx, ty, tz, coord_planar, npad, half_f):
    e = row.shape[0]
    per_tile = e // NS
    ce = 2000
    nch = per_tile // ce
    ngrp = ce // 16
    rows = half_f // 128
    rows_t = rows // NS
    slice_f = rows_t * 128
    mesh = plsc.VectorSubcoreMesh(core_axis_name="c", subcore_axis_name="s")

    @functools.partial(
        pl.kernel,
        out_type=jax.ShapeDtypeStruct((NC * half_f,), jnp.float32),
        mesh=mesh,
        compiler_params=pltpu.CompilerParams(needs_layout_passes=False),
        scratch_types=[
            pltpu.VMEM((rows, 128), jnp.float32),
            pltpu.VMEM((ce,), jnp.int32),
            pltpu.VMEM((ce,), jnp.float32),
            pltpu.VMEM((ce,), jnp.float32),
            pltpu.VMEM((ce,), jnp.float32),
            pltpu.VMEM((NS, rows_t, 128), jnp.float32),
            pltpu.VMEM((slice_f,), jnp.float32),
            pltpu.VMEM((slice_f,), jnp.float32),
            pltpu.VMEM_SHARED((NS, rows, 128), jnp.float32),
        ],
    )
    def scatter_k(row_hbm, tx_hbm, ty_hbm, tz_hbm, coord_hbm, out_hbm,
                  acc, ibuf, xbuf, ybuf, zbuf, redbuf, cbuf, obuf, shared):
        cid = lax.axis_index("c")
        sid = lax.axis_index("s")
        lo = cid * half_f

        def zero_body(i, c):
            for k in range(8):
                acc[i, pl.ds(k * 16, 16)] = jnp.zeros((16,), jnp.float32)
            return c

        lax.fori_loop(0, rows, zero_body, 0)

        def chunk_body(i, c):
            off = sid * per_tile + i * ce
            pltpu.sync_copy(row_hbm.at[pl.ds(off, ce)], ibuf)
            pltpu.sync_copy(tx_hbm.at[pl.ds(off, ce)], xbuf)
            pltpu.sync_copy(ty_hbm.at[pl.ds(off, ce)], ybuf)
            pltpu.sync_copy(tz_hbm.at[pl.ds(off, ce)], zbuf)

            def grp_body(gi, c2):
                sl = pl.ds(gi * 16, 16)
                iv = ibuf[sl]
                for ci, buf in ((0, xbuf), (1, ybuf), (2, zbuf)):
                    pv = iv + (ci * npad - lo)
                    inb = (pv >= 0) & (pv < half_f)
                    pvc = jnp.where(inb, pv, 0)
                    hi = lax.shift_right_logical(pvc, 7)
                    lo7 = jnp.bitwise_and(pvc, 127)
                    plsc.addupdate_scatter(acc, [hi, lo7], buf[sl], mask=inb)
                return c2

            lax.fori_loop(0, ngrp, grp_body, 0)
            return c

        lax.fori_loop(0, nch, chunk_body, 0)

        plsc.subcore_barrier()
        pltpu.sync_copy(acc, shared.at[sid])
        plsc.subcore_barrier()

        pltpu.sync_copy(shared.at[:, pl.ds(sid * rows_t, rows_t), :], redbuf)
        pltpu.sync_copy(coord_hbm.at[pl.ds(lo + sid * slice_f, slice_f)], cbuf)

        def red_body(gi, c):
            r = lax.shift_right_logical(gi, 3)
            k = jnp.bitwise_and(gi, 7)
            sl = pl.ds(k * 16, 16)
            fl = pl.ds(r * 128 + k * 16, 16)
            s = cbuf[fl]
            for t in range(NS):
                s = s + redbuf[t, r, sl]
            obuf[fl] = s
            return c

        lax.fori_loop(0, rows_t * 8, red_body, 0)
        pltpu.sync_copy(obuf, out_hbm.at[pl.ds(lo + sid * slice_f, slice_f)])

    return scatter_k(row, tx, ty, tz, coord_planar)


def kernel(h, coord, edge_index, coord_diff, coord_cross, edge_attr,
           w1, b1, w2, b2, w3):
    n, hdim = h.shape
    e = edge_index.shape[1]

    row = edge_index[0]
    col = edge_index[1]

    w1a = w1[:hdim]
    w1b = w1[hdim:2 * hdim]
    w1e = w1[2 * hdim:2 * hdim + 1]
    b1r = b1.reshape(1, hdim)
    b2r = b2.reshape(1, hdim)
    w3r = w3.reshape(1, hdim)

    atab, btab = _precompute_tables(h, w1a, w1b, b1r)

    cdt = coord_diff.T

    nblk = e // 2560
    cuts = [0, (nblk // 4) * 2560, (nblk // 2) * 2560,
            (3 * nblk // 4) * 2560, e]
    gs, ts = [], []
    for s in range(len(cuts) - 1):
        lo, hi = cuts[s], cuts[s + 1]
        gs.append(_sc_gather(row[lo:hi], col[lo:hi], atab, btab))
    for s in range(len(cuts) - 1):
        lo, hi = cuts[s], cuts[s + 1]
        ts.append(_mlp_edges(gs[s], edge_attr[lo:hi], cdt[:, lo:hi],
                             w1e, w2, b2r, w3r))
    trans_x = jnp.concatenate([t[0] for t in ts])
    trans_y = jnp.concatenate([t[1] for t in ts])
    trans_z = jnp.concatenate([t[2] for t in ts])

    npad = 10240
    half_f = 16384
    coord_planar = jnp.pad(coord.T, ((0, 0), (0, npad - n))).reshape(-1)
    coord_planar = jnp.pad(coord_planar, (0, NC * half_f - 3 * npad))
    out_flat = _sc_scatter(row, trans_x.reshape(-1), trans_y.reshape(-1),
                           trans_z.reshape(-1), coord_planar, npad, half_f)
    return out_flat[:3 * npad].reshape(3, npad)[:, :n].T
